# Initial kernel scaffold; baseline (speedup 1.0000x reference)
import jax
import jax.numpy as jnp
from jax.experimental import pallas as pl


def kernel(x, edge_index, edge_attr, Wni, bni, Wnj, bnj, We, be, attn_proj, Wmsg, bmsg, Wout, bout):
    raise NotImplementedError("write your pallas kernel here")


# speedup vs baseline: 1.0316x; 1.0316x over previous
import functools

import jax
import jax.numpy as jnp
from jax.experimental import pallas as pl
from jax.experimental.pallas import tpu as pltpu

N = 10000
E = 320000
D = 128
DE = 16
H = 8
HID = 16
HEAD = D // H


def _node_proj_body(x_ref, wni_ref, wnj_ref, wm1_ref, bni_ref, bnj_ref, bm_ref,
                    asrc_ref, adst_ref, mdst_ref):
    x = x_ref[...]
    asrc_ref[...] = jnp.dot(x, wni_ref[...], preferred_element_type=jnp.float32) + bni_ref[...]
    adst_ref[...] = jnp.dot(x, wnj_ref[...], preferred_element_type=jnp.float32) + bnj_ref[...]
    mdst_ref[...] = jnp.dot(x, wm1_ref[...], preferred_element_type=jnp.float32) + bm_ref[...]


def _edge_proj_body(ea_ref, we_ref, wm2_ref, be_ref, eh_ref, em_ref):
    ea = ea_ref[...]
    eh_ref[...] = jnp.dot(ea, we_ref[...], preferred_element_type=jnp.float32) + be_ref[...]
    em_ref[...] = jnp.dot(ea, wm2_ref[...], preferred_element_type=jnp.float32)


def _out_proj_body(agg_ref, wout_ref, bout_ref, out_ref):
    out_ref[...] = (jnp.dot(agg_ref[...], wout_ref[...], preferred_element_type=jnp.float32)
                    + bout_ref[...])


def _node_proj(x, Wni, Wnj, Wm1, bni, bnj, bm):
    bn = 2000
    grid = (N // bn,)
    full = pl.BlockSpec((D, D), lambda i: (0, 0))
    bias = pl.BlockSpec((1, D), lambda i: (0, 0))
    blk = pl.BlockSpec((bn, D), lambda i: (i, 0))
    return pl.pallas_call(
        _node_proj_body,
        grid=grid,
        in_specs=[blk, full, full, full, bias, bias, bias],
        out_specs=[blk, blk, blk],
        out_shape=[jax.ShapeDtypeStruct((N, D), jnp.float32)] * 3,
    )(x, Wni, Wnj, Wm1, bni.reshape(1, D), bnj.reshape(1, D), bm.reshape(1, D))


def _edge_proj(ea, We, Wm2, be):
    be_blk = 8000
    grid = (E // be_blk,)
    blk_in = pl.BlockSpec((be_blk, DE), lambda i: (i, 0))
    full = pl.BlockSpec((DE, D), lambda i: (0, 0))
    bias = pl.BlockSpec((1, D), lambda i: (0, 0))
    blk_out = pl.BlockSpec((be_blk, D), lambda i: (i, 0))
    return pl.pallas_call(
        _edge_proj_body,
        grid=grid,
        in_specs=[blk_in, full, full, bias],
        out_specs=[blk_out, blk_out],
        out_shape=[jax.ShapeDtypeStruct((E, D), jnp.float32)] * 2,
    )(ea, We, Wm2, be.reshape(1, D))


def _out_proj(agg, Wout, bout):
    bn = 2000
    grid = (N // bn,)
    blk = pl.BlockSpec((bn, D), lambda i: (i, 0))
    full = pl.BlockSpec((D, D), lambda i: (0, 0))
    bias = pl.BlockSpec((1, D), lambda i: (0, 0))
    return pl.pallas_call(
        _out_proj_body,
        grid=grid,
        in_specs=[blk, full, bias],
        out_specs=blk,
        out_shape=jax.ShapeDtypeStruct((N, D), jnp.float32),
    )(agg, Wout, bout.reshape(1, D))


def kernel(x, edge_index, edge_attr, Wni, bni, Wnj, bnj, We, be, attn_proj,
           Wmsg, bmsg, Wout, bout):
    src = edge_index[0].astype(jnp.int32)
    dst = edge_index[1].astype(jnp.int32)

    aSrc, aDst, mDst = _node_proj(x, Wni, Wnj, Wmsg[:D], bni, bnj, bmsg)
    eH, eM = _edge_proj(edge_attr, We, Wmsg[D:], be)

    u = aSrc[src] + aDst[dst] + eH
    u = jnp.maximum(u, 0.2 * u)
    logits = (u.reshape(-1, H, HID) * attn_proj).sum(-1)
    gmax = logits.max()
    ex = jnp.exp(logits - gmax)
    denom = jax.ops.segment_sum(ex, src, num_segments=N)
    attn = ex / (denom[src] + 1e-16)
    msg = (mDst[dst] + eM).reshape(-1, H, HEAD) * attn[..., None]
    agg = jax.ops.segment_sum(msg, src, num_segments=N).reshape(N, D)
    return _out_proj(agg, Wout, bout)


# speedup vs baseline: 1.1104x; 1.0764x over previous
import functools

import jax
import jax.numpy as jnp
from jax import lax
from jax.experimental import pallas as pl
from jax.experimental.pallas import tpu as pltpu
from jax.experimental.pallas import tpu_sc as plsc

N = 10000
E = 320000
D = 128
DE = 16
H = 8
HID = 16
HEAD = D // H

NC = 2
NS = 16
NW = NC * NS
WE = E // NW
C = 80
NCHUNK = WE // C

NP = 10240
NPT = NP // NS

NEG = -1e30

_SC_PARAMS = pltpu.CompilerParams(needs_layout_passes=False)


def _sc_mesh():
    return plsc.VectorSubcoreMesh(core_axis_name="c", subcore_axis_name="s")


def _node_proj_body(x_ref, wni_ref, wnj_ref, wm1_ref, bni_ref, bnj_ref, bm_ref,
                    asrc_ref, adst_ref, mdst_ref):
    x = x_ref[...]
    asrc_ref[...] = jnp.dot(x, wni_ref[...], preferred_element_type=jnp.float32) + bni_ref[...]
    adst_ref[...] = jnp.dot(x, wnj_ref[...], preferred_element_type=jnp.float32) + bnj_ref[...]
    mdst_ref[...] = jnp.dot(x, wm1_ref[...], preferred_element_type=jnp.float32) + bm_ref[...]


def _edge_proj_body(ea_ref, we_ref, wm2_ref, be_ref, eh_ref, em_ref):
    ea = ea_ref[...]
    eh_ref[...] = jnp.dot(ea, we_ref[...], preferred_element_type=jnp.float32) + be_ref[...]
    em_ref[...] = jnp.dot(ea, wm2_ref[...], preferred_element_type=jnp.float32)


def _out_proj_body(a0_ref, a1_ref, d0_ref, d1_ref, wout_ref, bout_ref, out_ref):
    agg = a0_ref[...] + a1_ref[...]
    den = (d0_ref[...] + d1_ref[...])[:, :H] + 1e-16
    rep = jnp.repeat(jnp.eye(H, dtype=jnp.float32), HID, axis=1)
    deno = jnp.dot(den, rep, preferred_element_type=jnp.float32)
    agg = agg / deno
    out_ref[...] = (jnp.dot(agg, wout_ref[...], preferred_element_type=jnp.float32)
                    + bout_ref[...])


def _node_proj(x, Wni, Wnj, Wm1, bni, bnj, bm):
    bn = 2000
    grid = (N // bn,)
    full = pl.BlockSpec((D, D), lambda i: (0, 0))
    bias = pl.BlockSpec((1, D), lambda i: (0, 0))
    blk = pl.BlockSpec((bn, D), lambda i: (i, 0))
    return pl.pallas_call(
        _node_proj_body,
        grid=grid,
        in_specs=[blk, full, full, full, bias, bias, bias],
        out_specs=[blk, blk, blk],
        out_shape=[jax.ShapeDtypeStruct((N, D), jnp.float32)] * 3,
    )(x, Wni, Wnj, Wm1, bni.reshape(1, D), bnj.reshape(1, D), bm.reshape(1, D))


def _edge_proj(ea, We, Wm2, be):
    be_blk = 8000
    grid = (E // be_blk,)
    blk_in = pl.BlockSpec((be_blk, DE), lambda i: (i, 0))
    full = pl.BlockSpec((DE, D), lambda i: (0, 0))
    bias = pl.BlockSpec((1, D), lambda i: (0, 0))
    blk_out = pl.BlockSpec((be_blk, D), lambda i: (i, 0))
    return pl.pallas_call(
        _edge_proj_body,
        grid=grid,
        in_specs=[blk_in, full, full, bias],
        out_specs=[blk_out, blk_out],
        out_shape=[jax.ShapeDtypeStruct((E, D), jnp.float32)] * 2,
    )(ea, We, Wm2, be.reshape(1, D))


def _out_proj(agg0, agg1, den0, den1, Wout, bout):
    bn = 2000
    grid = (N // bn,)
    blk = pl.BlockSpec((bn, D), lambda i: (i, 0))
    dblk = pl.BlockSpec((bn, HID), lambda i: (i, 0))
    full = pl.BlockSpec((D, D), lambda i: (0, 0))
    bias = pl.BlockSpec((1, D), lambda i: (0, 0))
    return pl.pallas_call(
        _out_proj_body,
        grid=grid,
        in_specs=[blk, blk, dblk, dblk, full, bias],
        out_specs=blk,
        out_shape=jax.ShapeDtypeStruct((N, D), jnp.float32),
    )(agg0, agg1, den0, den1, Wout, bout.reshape(1, D))


def _p1_body(asrc, adst, eh, src, dst, ap,
             logits_hbm, wmax_hbm,
             src_v, dst_v, a_v, b_v, e_v, lo_v, mx_v, ap_v,
             sem0, sem1):
    wid = lax.axis_index("s") * NC + lax.axis_index("c")
    pltpu.sync_copy(ap, ap_v)
    mx_v[...] = jnp.full((HID,), NEG, jnp.float32)

    def chunk(i, _):
        base = wid * WE + i * C
        pltpu.sync_copy(src.at[pl.ds(base, C)], src_v)
        pltpu.sync_copy(dst.at[pl.ds(base, C)], dst_v)
        cp_a = pltpu.async_copy(asrc.at[src_v], a_v, sem0)
        cp_b = pltpu.async_copy(adst.at[dst_v], b_v, sem1)
        pltpu.sync_copy(eh.at[pl.ds(base, C)], e_v)
        cp_a.wait()
        cp_b.wait()

        def edge(e, mx):
            lvec = jnp.full((HID,), NEG, jnp.float32)
            for h in range(H):
                u = (a_v[e, pl.ds(h * HID, HID)]
                     + b_v[e, pl.ds(h * HID, HID)]
                     + e_v[e, pl.ds(h * HID, HID)])
                u = jnp.maximum(u, 0.2 * u)
                s = jnp.sum(u * ap_v[h, :])
                lvec = jnp.where(lax.iota(jnp.int32, HID) == h, s, lvec)
            lo_v[e, :] = lvec
            return jnp.maximum(mx, lvec)

        mx_v[...] = lax.fori_loop(0, C, edge, mx_v[...])
        pltpu.sync_copy(lo_v, logits_hbm.at[pl.ds(base, C)])
        return ()

    lax.fori_loop(0, NCHUNK, chunk, ())
    pltpu.sync_copy(mx_v, wmax_hbm.at[wid])


def _p1(asrc, adst, eh, src, dst, ap):
    f = pl.kernel(
        _p1_body,
        out_type=[
            jax.ShapeDtypeStruct((E, HID), jnp.float32),
            jax.ShapeDtypeStruct((NW, HID), jnp.float32),
        ],
        mesh=_sc_mesh(),
        compiler_params=_SC_PARAMS,
        scratch_types=[
            pltpu.VMEM((C,), jnp.int32),
            pltpu.VMEM((C,), jnp.int32),
            pltpu.VMEM((C, D), jnp.float32),
            pltpu.VMEM((C, D), jnp.float32),
            pltpu.VMEM((C, D), jnp.float32),
            pltpu.VMEM((C, HID), jnp.float32),
            pltpu.VMEM((HID,), jnp.float32),
            pltpu.VMEM((H, HID), jnp.float32),
            pltpu.SemaphoreType.DMA,
            pltpu.SemaphoreType.DMA,
        ],
    )
    return f(asrc, adst, eh, src, dst, ap)


def _gmax_scalar(wm_v):
    m = wm_v[0, :]
    for w in range(1, NW):
        m = jnp.maximum(m, wm_v[w, :])
    return jnp.max(m)


def _p2_body(logits, src, wmax,
             den0, den1,
             src_v, lo_v, wm_v, den_sh):
    cid = lax.axis_index("c")
    sid = lax.axis_index("s")
    wid = sid * NC + cid
    pltpu.sync_copy(wmax, wm_v)
    gm = _gmax_scalar(wm_v)

    def zrow(r, _):
        lo_v[r, :] = jnp.zeros((HID,), jnp.float32)
        return ()

    lax.fori_loop(0, C, zrow, ())
    for j in range(NPT // C):
        pltpu.sync_copy(lo_v, den_sh.at[pl.ds(sid * NPT + j * C, C)])
    plsc.subcore_barrier()

    def chunk(i, _):
        base = wid * WE + i * C
        pltpu.sync_copy(src.at[pl.ds(base, C)], src_v)
        pltpu.sync_copy(logits.at[pl.ds(base, C)], lo_v)

        def edge(e, _):
            lo_v[e, :] = jnp.exp(lo_v[e, :] - gm)
            return ()

        lax.fori_loop(0, C, edge, ())
        pltpu.sync_copy(lo_v, den_sh.at[src_v], add=True)
        return ()

    lax.fori_loop(0, NCHUNK, chunk, ())
    plsc.subcore_barrier()

    for j in range(NPT // C):
        pltpu.sync_copy(den_sh.at[pl.ds(sid * NPT + j * C, C)], lo_v)

        @pl.when(cid == 0)
        def _():
            pltpu.sync_copy(lo_v, den0.at[pl.ds(sid * NPT + j * C, C)])

        @pl.when(cid == 1)
        def _():
            pltpu.sync_copy(lo_v, den1.at[pl.ds(sid * NPT + j * C, C)])


def _p2(logits, src, wmax):
    f = pl.kernel(
        _p2_body,
        out_type=[
            jax.ShapeDtypeStruct((NP, HID), jnp.float32),
            jax.ShapeDtypeStruct((NP, HID), jnp.float32),
        ],
        mesh=_sc_mesh(),
        compiler_params=_SC_PARAMS,
        scratch_types=[
            pltpu.VMEM((C,), jnp.int32),
            pltpu.VMEM((C, HID), jnp.float32),
            pltpu.VMEM((NW, HID), jnp.float32),
            pltpu.VMEM_SHARED((NP, HID), jnp.float32),
        ],
    )
    return f(logits, src, wmax)


def kernel(x, edge_index, edge_attr, Wni, bni, Wnj, bnj, We, be, attn_proj,
           Wmsg, bmsg, Wout, bout):
    src = edge_index[0].astype(jnp.int32)
    dst = edge_index[1].astype(jnp.int32)

    aSrc, aDst, mDst = _node_proj(x, Wni, Wnj, Wmsg[:D], bni, bnj, bmsg)
    eH, eM = _edge_proj(edge_attr, We, Wmsg[D:], be)

    logits16, wmax = _p1(aSrc, aDst, eH, src, dst, attn_proj)
    logits = logits16[:, :H]
    gmax = wmax.max()
    ex = jnp.exp(logits - gmax)
    den = jnp.pad(jax.ops.segment_sum(ex, src, num_segments=N),
                  ((0, 0), (0, HID - H)))
    aggU = jax.ops.segment_sum(
        (mDst[dst] + eM).reshape(-1, H, HEAD) * ex[..., None],
        src, num_segments=N).reshape(N, D)
    return _out_proj(aggU, jnp.zeros((N, D), jnp.float32),
                     den, jnp.zeros((N, HID), jnp.float32), Wout, bout)
